# baseline (speedup 1.0000x reference)
import functools

import jax
import jax.numpy as jnp
from jax import lax
from jax.experimental import pallas as pl
from jax.experimental.pallas import tpu as pltpu
from jax.experimental.pallas import tpu_sc as plsc

EMB_DIM = 32
BATCH = 16384
NC = 2
NS = 16
NW = NC * NS
B_PER_W = BATCH // NW
GCHUNK = 128
NCHUNK = B_PER_W // GCHUNK
L = 16


def _rsqrt16(x):
    i = lax.bitcast_convert_type(x, jnp.int32)
    y = lax.bitcast_convert_type(jnp.int32(0x5F3759DF) - (i >> 1), jnp.float32)
    half = jnp.float32(0.5)
    three_half = jnp.float32(1.5)
    for _ in range(3):
        y = y * (three_half - half * x * y * y)
    return y


def _make_kernel():
    mesh = plsc.VectorSubcoreMesh(core_axis_name="c", subcore_axis_name="s")

    @functools.partial(
        pl.kernel,
        mesh=mesh,
        compiler_params=pltpu.CompilerParams(
            needs_layout_passes=False, use_tc_tiling_on_sc=False),
        out_type=jax.ShapeDtypeStruct((BATCH,), jnp.float32),
        scratch_types=[
            pltpu.VMEM((B_PER_W,), jnp.int32),
            pltpu.VMEM((B_PER_W,), jnp.int32),
            pltpu.VMEM((B_PER_W, EMB_DIM), jnp.float32),
            pltpu.VMEM((B_PER_W, EMB_DIM), jnp.float32),
            pltpu.VMEM((B_PER_W,), jnp.float32),
            pltpu.SemaphoreType.DMA,
            pltpu.SemaphoreType.DMA,
            pltpu.SemaphoreType.DMA,
        ],
    )
    def cosine_kernel(uids_hbm, jids_hbm, utab_hbm, jtab_hbm, out_hbm,
                      uidx_v, jidx_v, urows_v, jrows_v, outv,
                      sem_i, sem_u, sem_j):
        wid = lax.axis_index("s") * NC + lax.axis_index("c")
        base = wid * B_PER_W

        ci_u = pltpu.async_copy(uids_hbm.at[pl.ds(base, B_PER_W)], uidx_v, sem_i)
        ci_j = pltpu.async_copy(jids_hbm.at[pl.ds(base, B_PER_W)], jidx_v, sem_i)
        ci_u.wait()
        ci_j.wait()

        ucopies = []
        jcopies = []
        for c in range(NCHUNK):
            sl = pl.ds(c * GCHUNK, GCHUNK)
            ucopies.append(pltpu.async_copy(
                utab_hbm.at[uidx_v.at[sl]], urows_v.at[sl], sem_u))
            jcopies.append(pltpu.async_copy(
                jtab_hbm.at[jidx_v.at[sl]], jrows_v.at[sl], sem_j))
        for cp in ucopies:
            cp.wait()
        for cp in jcopies:
            cp.wait()

        iota16 = lax.iota(jnp.int32, 16)
        zeros = jnp.zeros((L,), jnp.float32)
        eps = jnp.float32(1e-12)

        def group_body(g, carry):
            ridx = iota16 + g * L
            d = zeros
            uu = zeros
            jj = zeros
            for col in range(EMB_DIM):
                cvec = jnp.full((L,), col, jnp.int32)
                u = plsc.load_gather(urows_v, [ridx, cvec])
                v = plsc.load_gather(jrows_v, [ridx, cvec])
                d = d + u * v
                uu = uu + u * u
                jj = jj + v * v
            uu = jnp.maximum(uu, eps)
            jj = jnp.maximum(jj, eps)
            r = d * _rsqrt16(uu) * _rsqrt16(jj)
            outv[pl.ds(g * L, L)] = r
            return carry

        lax.fori_loop(0, B_PER_W // L, group_body, 0)

        pltpu.sync_copy(outv, out_hbm.at[pl.ds(base, B_PER_W)])

    return cosine_kernel


_kernel_call = _make_kernel()


def kernel(user_ids, joke_ids, user_table, joke_table):
    out = _kernel_call(user_ids, joke_ids, user_table, joke_table)
    return out.reshape(BATCH, 1)


# speedup vs baseline: 1.3638x; 1.3638x over previous
import functools

import jax
import jax.numpy as jnp
from jax import lax
from jax.experimental import pallas as pl
from jax.experimental.pallas import tpu as pltpu
from jax.experimental.pallas import tpu_sc as plsc

EMB_DIM = 32
BATCH = 16384
NC = 2
NS = 16
NW = NC * NS
B_PER_W = BATCH // NW
L = 16
W = 32
NWAVE = B_PER_W // W


def _rsqrt16(x):
    i = lax.bitcast_convert_type(x, jnp.int32)
    y = lax.bitcast_convert_type(jnp.int32(0x5F3759DF) - (i >> 1), jnp.float32)
    for _ in range(3):
        y = y * (jnp.float32(1.5) - jnp.float32(0.5) * x * y * y)
    return y


def _make_kernel():
    mesh = plsc.VectorSubcoreMesh(core_axis_name="c", subcore_axis_name="s")

    @functools.partial(
        pl.kernel,
        mesh=mesh,
        compiler_params=pltpu.CompilerParams(needs_layout_passes=False),
        out_type=jax.ShapeDtypeStruct((BATCH,), jnp.float32),
        scratch_types=[
            pltpu.VMEM((B_PER_W,), jnp.int32),
            pltpu.VMEM((B_PER_W,), jnp.int32),
            pltpu.VMEM((8 * W, EMB_DIM), jnp.float32),
            pltpu.VMEM((8 * W, EMB_DIM), jnp.float32),
            pltpu.VMEM((B_PER_W,), jnp.float32),
            pltpu.SemaphoreType.DMA,
            pltpu.SemaphoreType.DMA,
            pltpu.SemaphoreType.DMA,
        ],
    )
    def cosine_kernel(uids_hbm, jids_hbm, utab_hbm, jtab_hbm, out_hbm,
                      uidx_v, jidx_v, ublk_v, jblk_v, outv,
                      sem_i, sem_u, sem_j):
        wid = lax.axis_index("s") * NC + lax.axis_index("c")
        base = wid * B_PER_W

        c2 = pltpu.async_copy(uids_hbm.at[pl.ds(base, B_PER_W)], uidx_v, sem_i)
        c3 = pltpu.async_copy(jids_hbm.at[pl.ds(base, B_PER_W)], jidx_v, sem_i)
        c2.wait()
        c3.wait()

        iota16 = lax.iota(jnp.int32, 16)
        zeros = jnp.zeros((L,), jnp.float32)
        eps = jnp.float32(1e-12)

        def wave_body(w, carry):
            wbase = w * W

            for g in range(W // L):
                uvec0 = uidx_v[pl.ds(wbase + g * L, L)]
                jvec0 = jidx_v[pl.ds(wbase + g * L, L)]
                for k in range(L):
                    u = uvec0[k]
                    j = jvec0[k]
                    ub = pl.multiple_of((u >> 3) << 3, 8)
                    jb = pl.multiple_of((j >> 3) << 3, 8)
                    dst = pl.multiple_of((g * L + k) * 8, 8)
                    pltpu.async_copy(
                        utab_hbm.at[pl.ds(ub, 8), :],
                        ublk_v.at[pl.ds(dst, 8), :], sem_u)
                    pltpu.async_copy(
                        jtab_hbm.at[pl.ds(jb, 8), :],
                        jblk_v.at[pl.ds(dst, 8), :], sem_j)
            pltpu.make_async_copy(
                utab_hbm.at[pl.ds(0, 8 * W), :], ublk_v, sem_u).wait()
            pltpu.make_async_copy(
                jtab_hbm.at[pl.ds(0, 8 * W), :], jblk_v, sem_j).wait()

            for g in range(W // L):
                uvec = uidx_v[pl.ds(wbase + g * L, L)]
                jvec = jidx_v[pl.ds(wbase + g * L, L)]
                urow = (g * L + iota16) * 8 + (uvec & 7)
                jrow = (g * L + iota16) * 8 + (jvec & 7)
                d = zeros
                uu = zeros
                jj = zeros
                for f in range(EMB_DIM):
                    cvec = jnp.full((L,), f, jnp.int32)
                    uf = plsc.load_gather(ublk_v, [urow, cvec])
                    jf = plsc.load_gather(jblk_v, [jrow, cvec])
                    d = d + uf * jf
                    uu = uu + uf * uf
                    jj = jj + jf * jf
                uu = jnp.maximum(uu, eps)
                jj = jnp.maximum(jj, eps)
                outv[pl.ds(wbase + g * L, L)] = d * _rsqrt16(uu) * _rsqrt16(jj)
            return carry

        lax.fori_loop(0, NWAVE, wave_body, 0)
        pltpu.sync_copy(outv, out_hbm.at[pl.ds(base, B_PER_W)])

    return cosine_kernel


_kernel_call = _make_kernel()


def kernel(user_ids, joke_ids, user_table, joke_table):
    out = _kernel_call(user_ids, joke_ids, user_table, joke_table)
    return out.reshape(BATCH, 1)
